# baseline (speedup 1.0000x reference)
import functools

import jax
import jax.numpy as jnp
from jax import lax
from jax.experimental import pallas as pl
from jax.experimental.pallas import tpu as pltpu
from jax.experimental.pallas import tpu_sc as plsc

MF_DIM = 32
BATCH = 16384
NC = 2
NS = 16
NW = NC * NS
B_PER_W = BATCH // NW
CHUNK = 128
NCHUNK = B_PER_W // CHUNK


def _gmf_body(ui_hbm, ii_hbm, ut_hbm, it_hbm, wb_hbm, out_hbm,
              idx_u, idx_i, cu, ci, wb_v, out_v, sem0, sem1):
    c = lax.axis_index("c")
    s = lax.axis_index("s")
    wid = s * NC + c

    pltpu.sync_copy(ui_hbm.at[wid], idx_u)
    pltpu.sync_copy(ii_hbm.at[wid], idx_i)
    pltpu.sync_copy(wb_hbm, wb_v)

    sems = (sem0, sem1)

    def fire(d):
        sem = sems[d % 2]
        copies = []
        for j in range(NCHUNK):
            copies.append(pltpu.async_copy(
                ut_hbm.at[d].at[idx_u.at[j]],
                cu.at[d, pl.ds(j * CHUNK, CHUNK)], sem))
            copies.append(pltpu.async_copy(
                it_hbm.at[d].at[idx_i.at[j]],
                ci.at[d, pl.ds(j * CHUNK, CHUNK)], sem))
        return copies

    in_flight = {0: fire(0), 1: fire(1)}
    for d in range(MF_DIM):
        for cp in in_flight.pop(d):
            cp.wait()
        if d + 2 < MF_DIM:
            in_flight[d + 2] = fire(d + 2)

    ws = [plsc.load_gather(wb_v, [jnp.full((16,), d, jnp.int32)])
          for d in range(MF_DIM)]
    bv = plsc.load_gather(wb_v, [jnp.full((16,), MF_DIM, jnp.int32)])

    def g_body(g, carry):
        base = g * 16
        acc = bv
        for d in range(MF_DIM):
            u_d = cu[d, pl.ds(base, 16)]
            v_d = ci[d, pl.ds(base, 16)]
            acc = acc + u_d * v_d * ws[d]
        out_v[pl.ds(base, 16)] = 1.0 / (1.0 + jnp.exp(-acc))
        return carry

    lax.fori_loop(0, B_PER_W // 16, g_body, 0)
    pltpu.sync_copy(out_v, out_hbm.at[pl.ds(wid * B_PER_W, B_PER_W)])


@functools.partial(
    pl.kernel,
    mesh=plsc.VectorSubcoreMesh(core_axis_name="c", subcore_axis_name="s"),
    out_type=jax.ShapeDtypeStruct((BATCH,), jnp.float32),
    compiler_params=pltpu.CompilerParams(
        needs_layout_passes=False, use_tc_tiling_on_sc=False),
    scratch_types=[
        pltpu.VMEM((NCHUNK, CHUNK), jnp.int32),
        pltpu.VMEM((NCHUNK, CHUNK), jnp.int32),
        pltpu.VMEM((MF_DIM, B_PER_W), jnp.float32),
        pltpu.VMEM((MF_DIM, B_PER_W), jnp.float32),
        pltpu.VMEM((48,), jnp.float32),
        pltpu.VMEM((B_PER_W,), jnp.float32),
        pltpu.SemaphoreType.DMA,
        pltpu.SemaphoreType.DMA,
    ],
)
def _gmf_sc(*args):
    _gmf_body(*args)


def kernel(user_input, item_input, user_table, item_table, W, b):
    ui = user_input.astype(jnp.int32).reshape(NW, NCHUNK, CHUNK)
    ii = item_input.astype(jnp.int32).reshape(NW, NCHUNK, CHUNK)
    wb = jnp.concatenate([
        W.reshape(MF_DIM).astype(jnp.float32),
        b.reshape(1).astype(jnp.float32),
        jnp.zeros((15,), jnp.float32),
    ])
    out = _gmf_sc(ui, ii, user_table.T, item_table.T, wb)
    return out.reshape(BATCH, 1)


# speedup vs baseline: 18.5752x; 18.5752x over previous
import functools

import jax
import jax.numpy as jnp
from jax import lax
from jax.experimental import pallas as pl
from jax.experimental.pallas import tpu as pltpu
from jax.experimental.pallas import tpu_sc as plsc

MF_DIM = 32
BATCH = 16384
NC = 2
NS = 16
NW = NC * NS
B_PER_W = BATCH // NW
BLK = 128
WAVE = 16


def _gmf_body(ui_hbm, ii_hbm, ut_hbm, it_hbm, wb_hbm, out_hbm,
              idx_uv, idx_iv, blks, cu, ci, wb_v, out_v, sem):
    c = lax.axis_index("c")
    s = lax.axis_index("s")
    wid = s * NC + c

    pltpu.sync_copy(ui_hbm.at[wid], idx_uv)
    pltpu.sync_copy(ii_hbm.at[wid], idx_iv)
    pltpu.sync_copy(wb_hbm, wb_v)

    lanes = lax.iota(jnp.int32, 16)

    def wave(step, idx_ref, tab_hbm, dst):
        chunk = idx_ref[pl.ds(step * WAVE, WAVE)]
        rs = [chunk[lane] for lane in range(WAVE)]
        for lane in range(WAVE):
            base = pl.multiple_of((rs[lane] >> 7) * BLK, BLK)
            pltpu.async_copy(
                tab_hbm.at[:, pl.ds(base, BLK)], blks.at[lane], sem)
        for lane in range(WAVE):
            pltpu.make_async_copy(
                tab_hbm.at[:, pl.ds(0, BLK)], blks.at[lane], sem).wait()
        evec = step * WAVE + lanes
        for lane in range(WAVE):
            col = jnp.full((16,), rs[lane] & (BLK - 1), jnp.int32)
            ev = jnp.full((16,), step * WAVE + lane, jnp.int32)
            for h in range(2):
                dvec = lanes + 16 * h
                x = plsc.load_gather(blks.at[lane], [dvec, col])
                plsc.store_scatter(dst, [dvec, ev], x)
        del evec

    def w_body(step, carry):
        wave(step, idx_uv, ut_hbm, cu)
        wave(step, idx_iv, it_hbm, ci)
        return carry

    lax.fori_loop(0, B_PER_W // WAVE, w_body, 0)

    ws = [plsc.load_gather(wb_v, [jnp.full((16,), d, jnp.int32)])
          for d in range(MF_DIM)]
    bv = plsc.load_gather(wb_v, [jnp.full((16,), MF_DIM, jnp.int32)])

    def g_body(g, carry):
        rows = g * 16 + lanes
        acc = bv
        for d in range(MF_DIM):
            dcol = jnp.full((16,), d, jnp.int32)
            u_d = plsc.load_gather(cu, [dcol, rows])
            v_d = plsc.load_gather(ci, [dcol, rows])
            acc = acc + u_d * v_d * ws[d]
        out_v[pl.ds(g * 16, 16)] = 1.0 / (1.0 + jnp.exp(-acc))
        return carry

    lax.fori_loop(0, B_PER_W // 16, g_body, 0)
    pltpu.sync_copy(out_v, out_hbm.at[pl.ds(wid * B_PER_W, B_PER_W)])


@functools.partial(
    pl.kernel,
    mesh=plsc.VectorSubcoreMesh(core_axis_name="c", subcore_axis_name="s"),
    out_type=jax.ShapeDtypeStruct((BATCH,), jnp.float32),
    compiler_params=pltpu.CompilerParams(
        needs_layout_passes=False, use_tc_tiling_on_sc=True),
    scratch_types=[
        pltpu.VMEM((B_PER_W,), jnp.int32),
        pltpu.VMEM((B_PER_W,), jnp.int32),
        pltpu.VMEM((WAVE, MF_DIM, BLK), jnp.float32),
        pltpu.VMEM((MF_DIM, B_PER_W), jnp.float32),
        pltpu.VMEM((MF_DIM, B_PER_W), jnp.float32),
        pltpu.VMEM((48,), jnp.float32),
        pltpu.VMEM((B_PER_W,), jnp.float32),
        pltpu.SemaphoreType.DMA,
    ],
)
def _gmf_sc(*args):
    _gmf_body(*args)


def kernel(user_input, item_input, user_table, item_table, W, b):
    ui = user_input.astype(jnp.int32).reshape(NW, B_PER_W)
    ii = item_input.astype(jnp.int32).reshape(NW, B_PER_W)
    wb = jnp.concatenate([
        W.reshape(MF_DIM).astype(jnp.float32),
        b.reshape(1).astype(jnp.float32),
        jnp.zeros((15,), jnp.float32),
    ])
    out = _gmf_sc(ui, ii, user_table.T, item_table.T, wb)
    return out.reshape(BATCH, 1)
